# baseline (speedup 1.0000x reference)
import functools

import jax
import jax.numpy as jnp
from jax import lax
from jax.experimental import pallas as pl
from jax.experimental.pallas import tpu as pltpu
from jax.experimental.pallas import tpu_sc as plsc

EPS = 1e-07
INV_TEMP = 10.0

NC, NS = 2, 16
NW = NC * NS
IDX_W = 125
CH_ROWS = 8
CH = IDX_W * CH_ROWS


def _make_sc_gather(d, E):
    EPW = E // NW
    n_chunks = EPW // CH
    ir_pt = EPW // IDX_W
    mesh = plsc.VectorSubcoreMesh(core_axis_name="c", subcore_axis_name="s")

    @functools.partial(
        pl.kernel,
        out_type=jax.ShapeDtypeStruct((E, d), jnp.float32),
        mesh=mesh,
        scratch_types=[
            pltpu.VMEM((ir_pt, IDX_W), jnp.int32),
            pltpu.VMEM((CH, d), jnp.float32),
            pltpu.VMEM((CH, d), jnp.float32),
            pltpu.SemaphoreType.DMA,
            pltpu.SemaphoreType.DMA,
            pltpu.SemaphoreType.DMA,
            pltpu.SemaphoreType.DMA,
        ],
        compiler_params=pltpu.CompilerParams(use_tc_tiling_on_sc=False),
    )
    def gk(table_hbm, idx_hbm, out_hbm, idx_v, rows_a, rows_b,
           gsem_a, gsem_b, wsem_a, wsem_b):
        c = lax.axis_index("c")
        s = lax.axis_index("s")
        wid = c * NS + s
        pltpu.sync_copy(idx_hbm.at[pl.ds(wid * ir_pt, ir_pt)], idx_v)

        def fire8(k, rows, sem):
            for j in range(CH_ROWS):
                pltpu.async_copy(table_hbm.at[idx_v.at[k * CH_ROWS + j]],
                                 rows.at[pl.ds(j * IDX_W, IDX_W)], sem)

        def drain8(k, rows, sem):
            for j in range(CH_ROWS):
                pltpu.make_async_copy(
                    table_hbm.at[idx_v.at[k * CH_ROWS + j]],
                    rows.at[pl.ds(j * IDX_W, IDX_W)], sem).wait()

        def fire_w(k, rows, sem):
            pltpu.async_copy(rows, out_hbm.at[pl.ds(wid * EPW + k * CH, CH)],
                             sem)

        def drain_w(k, rows, sem):
            pltpu.make_async_copy(
                rows, out_hbm.at[pl.ds(wid * EPW + k * CH, CH)], sem).wait()

        def half(k, first):
            if not first:
                drain_w(k - 1, rows_b, wsem_b)
            fire8(k + 1, rows_b, gsem_b)
            drain8(k, rows_a, gsem_a)
            fire_w(k, rows_a, wsem_a)
            drain_w(k, rows_a, wsem_a)
            fire8(k + 2, rows_a, gsem_a)
            drain8(k + 1, rows_b, gsem_b)
            fire_w(k + 1, rows_b, wsem_b)

        fire8(0, rows_a, gsem_a)
        half(0, True)

        def pair(j, carry):
            half(2 * j, False)
            return carry

        lax.fori_loop(1, (n_chunks - 1) // 2, pair, 0)
        drain_w(n_chunks - 2, rows_b, wsem_b)
        drain8(n_chunks - 1, rows_a, gsem_a)
        fire_w(n_chunks - 1, rows_a, wsem_a)
        drain_w(n_chunks - 1, rows_a, wsem_a)

    return gk


def _make_sc_scatter(n_nodes, E, d):
    E_half = E // NC
    EPW = E_half // NS
    ch_rows = 2
    ch = ch_rows * IDX_W
    n_chunks = EPW // ch
    rpt = n_nodes // NS
    mesh = plsc.VectorSubcoreMesh(core_axis_name="c", subcore_axis_name="s")

    @functools.partial(
        pl.kernel,
        out_type=jax.ShapeDtypeStruct((NC * n_nodes, d), jnp.float32),
        mesh=mesh,
        scratch_types=[
            pltpu.VMEM((ch_rows, IDX_W), jnp.int32),
            pltpu.VMEM((ch, d), jnp.float32),
            pltpu.VMEM((ch_rows, IDX_W), jnp.int32),
            pltpu.VMEM((ch, d), jnp.float32),
            pltpu.VMEM_SHARED((n_nodes, d), jnp.float32),
            pltpu.SemaphoreType.DMA,
            pltpu.SemaphoreType.DMA,
            pltpu.SemaphoreType.DMA,
            pltpu.SemaphoreType.DMA,
        ],
        compiler_params=pltpu.CompilerParams(use_tc_tiling_on_sc=False),
    )
    def sk(vals_hbm, idx_hbm, zeros_hbm, out_hbm, idx_a, rows_a, idx_b,
           rows_b, acc_sh, lsem_a, lsem_b, ssem_a, ssem_b):
        c = lax.axis_index("c")
        s = lax.axis_index("s")
        pltpu.sync_copy(zeros_hbm.at[pl.ds(s * rpt, rpt)],
                        acc_sh.at[pl.ds(s * rpt, rpt)])
        plsc.subcore_barrier()
        rbase0 = (c * E_half + s * EPW) // IDX_W
        ebase0 = c * E_half + s * EPW

        def fire_loads(k, idx_v, rows_v, sem):
            pltpu.async_copy(idx_hbm.at[pl.ds(rbase0 + k * ch_rows, ch_rows)],
                             idx_v, sem)
            pltpu.async_copy(vals_hbm.at[pl.ds(ebase0 + k * ch, ch)],
                             rows_v, sem)

        def drain_loads(k, idx_v, rows_v, sem):
            pltpu.make_async_copy(
                idx_hbm.at[pl.ds(rbase0 + k * ch_rows, ch_rows)], idx_v,
                sem).wait()
            pltpu.make_async_copy(
                vals_hbm.at[pl.ds(ebase0 + k * ch, ch)], rows_v, sem).wait()

        def fire_sc(idx_v, rows_v, sem):
            for j in range(ch_rows):
                pltpu.async_copy(rows_v.at[pl.ds(j * IDX_W, IDX_W)],
                                 acc_sh.at[idx_v.at[j]], sem, add=True)

        def drain_sc(idx_v, rows_v, sem):
            for j in range(ch_rows):
                pltpu.make_async_copy(
                    rows_v.at[pl.ds(j * IDX_W, IDX_W)],
                    acc_sh.at[idx_v.at[j]], sem).wait()

        def half(k, first):
            if not first:
                drain_sc(idx_b, rows_b, ssem_b)
            drain_loads(k, idx_a, rows_a, lsem_a)
            fire_loads(k + 1, idx_b, rows_b, lsem_b)
            fire_sc(idx_a, rows_a, ssem_a)
            drain_loads(k + 1, idx_b, rows_b, lsem_b)
            drain_sc(idx_a, rows_a, ssem_a)
            fire_loads(k + 2, idx_a, rows_a, lsem_a)
            fire_sc(idx_b, rows_b, ssem_b)

        fire_loads(0, idx_a, rows_a, lsem_a)
        half(0, True)

        def pair(j, carry):
            half(2 * j, False)
            return carry

        lax.fori_loop(1, (n_chunks - 2) // 2, pair, 0)
        drain_sc(idx_b, rows_b, ssem_b)
        drain_loads(n_chunks - 2, idx_a, rows_a, lsem_a)
        fire_loads(n_chunks - 1, idx_b, rows_b, lsem_b)
        fire_sc(idx_a, rows_a, ssem_a)
        drain_loads(n_chunks - 1, idx_b, rows_b, lsem_b)
        drain_sc(idx_a, rows_a, ssem_a)
        fire_sc(idx_b, rows_b, ssem_b)
        drain_sc(idx_b, rows_b, ssem_b)
        plsc.subcore_barrier()
        pltpu.sync_copy(acc_sh.at[pl.ds(s * rpt, rpt)],
                        out_hbm.at[pl.ds(c * n_nodes + s * rpt, rpt)])

    return sk


_ROT = ((13, 15, 26, 6), (17, 29, 16, 24))


def _uniform_at(kd_ref, shape, row0, stride, col0):
    k1 = kd_ref[0]
    k2 = kd_ref[1]
    ks = [k1, k2, k1 ^ k2 ^ jnp.uint32(0x1BD11BDA)]
    row = lax.broadcasted_iota(jnp.uint32, shape, 0)
    col = lax.broadcasted_iota(jnp.uint32, shape, 1)
    p = (row + row0.astype(jnp.uint32)) * jnp.uint32(stride) \
        + col + jnp.uint32(col0)
    x0 = jnp.zeros(shape, jnp.uint32) + ks[0]
    x1 = p + ks[1]
    for blk in range(5):
        for r in _ROT[blk % 2]:
            x0 = x0 + x1
            x1 = ((x1 << jnp.uint32(r)) | (x1 >> jnp.uint32(32 - r))) ^ x0
        x0 = x0 + ks[(blk + 1) % 3]
        x1 = x1 + ks[(blk + 2) % 3] + jnp.uint32(blk + 1)
    bits = x0 ^ x1
    fb = (bits >> jnp.uint32(9)) | jnp.uint32(0x3F800000)
    return jnp.maximum(lax.bitcast_convert_type(fb, jnp.float32) - 1.0, 0.0)

def _scale(u, a10, ir):
    up = u + EPS
    um = 1.0 - u + EPS
    a2 = up * up
    a4 = a2 * a2
    a = a4 * a4 * a2
    b2 = um * um
    b4 = b2 * b2
    b = b4 * b4 * b2
    return ir * b / (a10 * a + b)


def _rng_scale_body(sc_ref, kd_ref, o_ref):
    a10, ir = sc_ref[0], sc_ref[1]
    br, wd = o_ref.shape
    u = _uniform_at(kd_ref, (br, wd), pl.program_id(0) * br, wd, 0)
    o_ref[...] = _scale(u, a10, ir)


def _tc_rng_scale(sc, kd, n):
    wd = 128
    rows = n // wd
    br = 1000 if rows % 1000 == 0 else rows
    return pl.pallas_call(
        _rng_scale_body,
        grid=(rows // br,),
        in_specs=[_SMEM_SPEC, _SMEM_SPEC],
        out_specs=pl.BlockSpec((br, wd), lambda i: (i, 0)),
        out_shape=jax.ShapeDtypeStruct((rows, wd), jnp.float32),
    )(sc, kd)


def _init_body(s_ref, xg_ref, ea_ref, wx_ref, we_ref, o_ref):
    nd = xg_ref.shape[1]
    s = s_ref[...]
    a = (xg_ref[...] * s[:, :nd]) @ wx_ref[...]
    b = (ea_ref[...] * s[:, nd:]) @ we_ref[...]
    o_ref[...] = jnp.maximum(a + b, 0.0)


def _add_body(a_ref, b_ref, o_ref):
    o_ref[...] = a_ref[...] + b_ref[...]


def _step_body(s_ref, t_ref, e_ref, e0_ref, w4_ref, o_ref):
    e = e_ref[...]
    d4 = e.shape[1]
    h = d4 // 4
    rm = jnp.concatenate([e[:, h:2 * h], e[:, :h],
                          e[:, 3 * h:], e[:, 2 * h:3 * h]], axis=1)
    m = (t_ref[...] - rm) * s_ref[...]
    o_ref[...] = jnp.maximum(e0_ref[...] + m @ w4_ref[...], 0.0)


def _final_body(s_ref, x_ref, pa_ref, pb_ref, wx_ref, wf_ref, o_ref):
    nd = x_ref.shape[1]
    s = s_ref[...]
    ff = pa_ref[...] + pb_ref[...]
    a = (x_ref[...] * s[:, :nd]) @ wx_ref[...]
    b = (ff * s[:, nd:]) @ wf_ref[...]
    o_ref[...] = jnp.maximum(a + b, 0.0)


def _rows_spec(b, d):
    return pl.BlockSpec((b, d), lambda i: (i, 0))


def _full_spec(shape):
    return pl.BlockSpec(shape, lambda i: tuple(0 for _ in shape))


_SMEM_SPEC = pl.BlockSpec(memory_space=pltpu.SMEM)


def _tc_init(s0, xg, ea, wx, we):
    E, nd = xg.shape
    ed = ea.shape[1]
    eh = wx.shape[1]
    B = 8000
    return pl.pallas_call(
        _init_body,
        grid=(E // B,),
        in_specs=[_rows_spec(B, nd + ed), _rows_spec(B, nd),
                  _rows_spec(B, ed), _full_spec((nd, eh)),
                  _full_spec((ed, eh))],
        out_specs=_rows_spec(B, eh),
        out_shape=jax.ShapeDtypeStruct((E, eh), jnp.float32),
    )(s0, xg, ea, wx, we)


def _tc_add(a, b):
    n, d = a.shape
    B = 5000
    return pl.pallas_call(
        _add_body,
        grid=(n // B,),
        in_specs=[_rows_spec(B, d), _rows_spec(B, d)],
        out_specs=_rows_spec(B, d),
        out_shape=jax.ShapeDtypeStruct((n, d), jnp.float32),
    )(a, b)


def _tc_step(s4, t4, e4, e04, w4):
    r, d4 = t4.shape
    B = 4000
    return pl.pallas_call(
        _step_body,
        grid=(r // B,),
        in_specs=[_rows_spec(B, d4), _rows_spec(B, d4),
                  _rows_spec(B, d4), _rows_spec(B, d4), _full_spec((d4, d4))],
        out_specs=_rows_spec(B, d4),
        out_shape=jax.ShapeDtypeStruct((r, d4), jnp.float32),
    )(s4, t4, e4, e04, w4)


def _tc_final(sl, x, pa, pb, wx, wf):
    n, nd = x.shape
    eh = pa.shape[1]
    nh = wx.shape[1]
    B = 5000
    return pl.pallas_call(
        _final_body,
        grid=(n // B,),
        in_specs=[_rows_spec(B, nd + eh), _rows_spec(B, nd),
                  _rows_spec(B, eh), _rows_spec(B, eh),
                  _full_spec((nd, nh)), _full_spec((eh, nh))],
        out_specs=_rows_spec(B, nh),
        out_shape=jax.ShapeDtypeStruct((n, nh), jnp.float32),
    )(sl, x, pa, pb, wx, wf)


def kernel(x, edge_attr, edge_index, W_init, W_eupd, W_last,
           p_init, p_eupd, p_last):
    n_nodes, nd = x.shape
    E, ed = edge_attr.shape
    eh = W_eupd.shape[0]
    n_steps = 3

    src = edge_index[0].astype(jnp.int32)
    dst = edge_index[1].astype(jnp.int32)
    src2d = src.reshape(E // IDX_W, IDX_W)
    dst2d = dst.reshape(E // IDX_W, IDX_W)

    nk = jax.random.key(1)

    def kd(i):
        return jax.random.key_data(jax.random.fold_in(nk, i))

    def scpair(p_logit):
        p = jax.nn.sigmoid(p_logit[0])
        a10 = ((p + EPS) / (1.0 - p + EPS)) ** 10
        ir = 1.0 / (1.0 - p)
        return jnp.stack([a10, ir]).astype(jnp.float32)

    sc0, sce, scl = scpair(p_init), scpair(p_eupd), scpair(p_last)

    zeros_n = jnp.zeros((n_nodes, eh), jnp.float32)
    zW = jnp.zeros_like(W_eupd)
    w4 = jnp.block([[W_eupd, zW, zW, zW], [zW, W_eupd, zW, zW],
                    [zW, zW, W_eupd, zW],
                    [zW, zW, zW, W_eupd]])

    gather_x = _make_sc_gather(nd, E)
    gather_h = _make_sc_gather(eh, E)
    scatter = _make_sc_scatter(n_nodes, E, eh)

    s0 = _tc_rng_scale(sc0, kd(0), E * (nd + ed)).reshape(E, nd + ed)
    sl = _tc_rng_scale(scl, kd(99),
                       n_nodes * (nd + eh)).reshape(n_nodes, nd + eh)
    ss = [_tc_rng_scale(sce, kd(10 + i), E * eh) for i in range(n_steps)]

    xg = gather_x(x, src2d)
    e0 = _tc_init(s0, xg, edge_attr, W_init[:nd], W_init[nd:])
    e0r = e0.reshape(E // 4, 4 * eh)
    e_t = e0
    for i in range(n_steps):
        parts = scatter(e_t, dst2d, zeros_n)
        neigh = _tc_add(parts[:n_nodes], parts[n_nodes:])
        t = gather_h(neigh, src2d)
        e_t = _tc_step(ss[i], t.reshape(E // 4, 4 * eh),
                       e_t.reshape(E // 4, 4 * eh), e0r, w4)
        e_t = e_t.reshape(E, eh)
    parts = scatter(e_t, dst2d, zeros_n)
    return _tc_final(sl, x, parts[:n_nodes], parts[n_nodes:],
                     W_last[:nd], W_last[nd:])


# speedup vs baseline: 1.0284x; 1.0284x over previous
import functools

import jax
import jax.numpy as jnp
from jax import lax
from jax.experimental import pallas as pl
from jax.experimental.pallas import tpu as pltpu
from jax.experimental.pallas import tpu_sc as plsc

EPS = 1e-07
INV_TEMP = 10.0

NC, NS = 2, 16
NW = NC * NS
IDX_W = 125
CH_ROWS = 8
CH = IDX_W * CH_ROWS


def _make_sc_gather(d, E):
    EPW = E // NW
    n_chunks = EPW // CH
    ir_pt = EPW // IDX_W
    mesh = plsc.VectorSubcoreMesh(core_axis_name="c", subcore_axis_name="s")

    @functools.partial(
        pl.kernel,
        out_type=jax.ShapeDtypeStruct((E, d), jnp.float32),
        mesh=mesh,
        scratch_types=[
            pltpu.VMEM((ir_pt, IDX_W), jnp.int32),
            pltpu.VMEM((CH, d), jnp.float32),
            pltpu.VMEM((CH, d), jnp.float32),
            pltpu.SemaphoreType.DMA,
            pltpu.SemaphoreType.DMA,
            pltpu.SemaphoreType.DMA,
            pltpu.SemaphoreType.DMA,
        ],
        compiler_params=pltpu.CompilerParams(use_tc_tiling_on_sc=False),
    )
    def gk(table_hbm, idx_hbm, out_hbm, idx_v, rows_a, rows_b,
           gsem_a, gsem_b, wsem_a, wsem_b):
        c = lax.axis_index("c")
        s = lax.axis_index("s")
        wid = c * NS + s
        pltpu.sync_copy(idx_hbm.at[pl.ds(wid * ir_pt, ir_pt)], idx_v)

        def fire8(k, rows, sem):
            for j in range(CH_ROWS):
                pltpu.async_copy(table_hbm.at[idx_v.at[k * CH_ROWS + j]],
                                 rows.at[pl.ds(j * IDX_W, IDX_W)], sem)

        def drain8(k, rows, sem):
            for j in range(CH_ROWS):
                pltpu.make_async_copy(
                    table_hbm.at[idx_v.at[k * CH_ROWS + j]],
                    rows.at[pl.ds(j * IDX_W, IDX_W)], sem).wait()

        def fire_w(k, rows, sem):
            pltpu.async_copy(rows, out_hbm.at[pl.ds(wid * EPW + k * CH, CH)],
                             sem)

        def drain_w(k, rows, sem):
            pltpu.make_async_copy(
                rows, out_hbm.at[pl.ds(wid * EPW + k * CH, CH)], sem).wait()

        def half(k, first):
            if not first:
                drain_w(k - 1, rows_b, wsem_b)
            fire8(k + 1, rows_b, gsem_b)
            drain8(k, rows_a, gsem_a)
            fire_w(k, rows_a, wsem_a)
            drain_w(k, rows_a, wsem_a)
            fire8(k + 2, rows_a, gsem_a)
            drain8(k + 1, rows_b, gsem_b)
            fire_w(k + 1, rows_b, wsem_b)

        fire8(0, rows_a, gsem_a)
        half(0, True)

        def pair(j, carry):
            half(2 * j, False)
            return carry

        lax.fori_loop(1, (n_chunks - 1) // 2, pair, 0)
        drain_w(n_chunks - 2, rows_b, wsem_b)
        drain8(n_chunks - 1, rows_a, gsem_a)
        fire_w(n_chunks - 1, rows_a, wsem_a)
        drain_w(n_chunks - 1, rows_a, wsem_a)

    return gk


def _make_sc_scatter(n_nodes, E, d):
    E_half = E // NC
    EPW = E_half // NS
    ch_rows = 2
    ch = ch_rows * IDX_W
    n_chunks = EPW // ch
    rpt = n_nodes // NS
    mesh = plsc.VectorSubcoreMesh(core_axis_name="c", subcore_axis_name="s")

    @functools.partial(
        pl.kernel,
        out_type=jax.ShapeDtypeStruct((NC * n_nodes, d), jnp.float32),
        mesh=mesh,
        scratch_types=[
            pltpu.VMEM((ch_rows, IDX_W), jnp.int32),
            pltpu.VMEM((ch, d), jnp.float32),
            pltpu.VMEM((ch_rows, IDX_W), jnp.int32),
            pltpu.VMEM((ch, d), jnp.float32),
            pltpu.VMEM_SHARED((n_nodes, d), jnp.float32),
            pltpu.SemaphoreType.DMA,
            pltpu.SemaphoreType.DMA,
            pltpu.SemaphoreType.DMA,
            pltpu.SemaphoreType.DMA,
        ],
        compiler_params=pltpu.CompilerParams(use_tc_tiling_on_sc=False),
    )
    def sk(vals_hbm, idx_hbm, zeros_hbm, out_hbm, idx_a, rows_a, idx_b,
           rows_b, acc_sh, lsem_a, lsem_b, ssem_a, ssem_b):
        c = lax.axis_index("c")
        s = lax.axis_index("s")
        pltpu.sync_copy(zeros_hbm.at[pl.ds(s * rpt, rpt)],
                        acc_sh.at[pl.ds(s * rpt, rpt)])
        plsc.subcore_barrier()
        rbase0 = (c * E_half + s * EPW) // IDX_W
        ebase0 = c * E_half + s * EPW

        def fire_loads(k, idx_v, rows_v, sem):
            pltpu.async_copy(idx_hbm.at[pl.ds(rbase0 + k * ch_rows, ch_rows)],
                             idx_v, sem)
            pltpu.async_copy(vals_hbm.at[pl.ds(ebase0 + k * ch, ch)],
                             rows_v, sem)

        def drain_loads(k, idx_v, rows_v, sem):
            pltpu.make_async_copy(
                idx_hbm.at[pl.ds(rbase0 + k * ch_rows, ch_rows)], idx_v,
                sem).wait()
            pltpu.make_async_copy(
                vals_hbm.at[pl.ds(ebase0 + k * ch, ch)], rows_v, sem).wait()

        def fire_sc(idx_v, rows_v, sem):
            for j in range(ch_rows):
                pltpu.async_copy(rows_v.at[pl.ds(j * IDX_W, IDX_W)],
                                 acc_sh.at[idx_v.at[j]], sem, add=True)

        def drain_sc(idx_v, rows_v, sem):
            for j in range(ch_rows):
                pltpu.make_async_copy(
                    rows_v.at[pl.ds(j * IDX_W, IDX_W)],
                    acc_sh.at[idx_v.at[j]], sem).wait()

        def half(k, first):
            if not first:
                drain_sc(idx_b, rows_b, ssem_b)
            drain_loads(k, idx_a, rows_a, lsem_a)
            fire_loads(k + 1, idx_b, rows_b, lsem_b)
            fire_sc(idx_a, rows_a, ssem_a)
            drain_loads(k + 1, idx_b, rows_b, lsem_b)
            drain_sc(idx_a, rows_a, ssem_a)
            fire_loads(k + 2, idx_a, rows_a, lsem_a)
            fire_sc(idx_b, rows_b, ssem_b)

        fire_loads(0, idx_a, rows_a, lsem_a)
        half(0, True)

        def pair(j, carry):
            half(2 * j, False)
            return carry

        lax.fori_loop(1, (n_chunks - 2) // 2, pair, 0)
        drain_sc(idx_b, rows_b, ssem_b)
        drain_loads(n_chunks - 2, idx_a, rows_a, lsem_a)
        fire_loads(n_chunks - 1, idx_b, rows_b, lsem_b)
        fire_sc(idx_a, rows_a, ssem_a)
        drain_loads(n_chunks - 1, idx_b, rows_b, lsem_b)
        drain_sc(idx_a, rows_a, ssem_a)
        fire_sc(idx_b, rows_b, ssem_b)
        drain_sc(idx_b, rows_b, ssem_b)
        plsc.subcore_barrier()
        pltpu.sync_copy(acc_sh.at[pl.ds(s * rpt, rpt)],
                        out_hbm.at[pl.ds(c * n_nodes + s * rpt, rpt)])

    return sk


_ROT = ((13, 15, 26, 6), (17, 29, 16, 24))


def _uniform_at(kd_ref, shape, row0, stride, col0):
    k1 = kd_ref[0]
    k2 = kd_ref[1]
    ks = [k1, k2, k1 ^ k2 ^ jnp.uint32(0x1BD11BDA)]
    row = lax.broadcasted_iota(jnp.uint32, shape, 0)
    col = lax.broadcasted_iota(jnp.uint32, shape, 1)
    p = (row + row0.astype(jnp.uint32)) * jnp.uint32(stride) \
        + col + jnp.uint32(col0)
    x0 = jnp.zeros(shape, jnp.uint32) + ks[0]
    x1 = p + ks[1]
    for blk in range(5):
        for r in _ROT[blk % 2]:
            x0 = x0 + x1
            x1 = ((x1 << jnp.uint32(r)) | (x1 >> jnp.uint32(32 - r))) ^ x0
        x0 = x0 + ks[(blk + 1) % 3]
        x1 = x1 + ks[(blk + 2) % 3] + jnp.uint32(blk + 1)
    bits = x0 ^ x1
    fb = (bits >> jnp.uint32(9)) | jnp.uint32(0x3F800000)
    return jnp.maximum(lax.bitcast_convert_type(fb, jnp.float32) - 1.0, 0.0)

def _scale(u, a10, ir):
    up = u + EPS
    um = 1.0 - u + EPS
    a2 = up * up
    a4 = a2 * a2
    a = a4 * a4 * a2
    b2 = um * um
    b4 = b2 * b2
    b = b4 * b4 * b2
    return ir * b / (a10 * a + b)


def _rng_scale_body(sc_ref, kd_ref, o_ref):
    a10, ir = sc_ref[0], sc_ref[1]
    br, wd = o_ref.shape
    u = _uniform_at(kd_ref, (br, wd), pl.program_id(0) * br, wd, 0)
    o_ref[...] = _scale(u, a10, ir)


def _tc_rng_scale(sc, kd, n):
    wd = 128
    rows = n // wd
    br = 1000 if rows % 1000 == 0 else rows
    return pl.pallas_call(
        _rng_scale_body,
        grid=(rows // br,),
        in_specs=[_SMEM_SPEC, _SMEM_SPEC],
        out_specs=pl.BlockSpec((br, wd), lambda i: (i, 0)),
        out_shape=jax.ShapeDtypeStruct((rows, wd), jnp.float32),
    )(sc, kd)


def _init_body(s_ref, xg_ref, ea_ref, wx_ref, we_ref, o_ref):
    nd = xg_ref.shape[1]
    s = s_ref[...]
    a = (xg_ref[...] * s[:, :nd]) @ wx_ref[...]
    b = (ea_ref[...] * s[:, nd:]) @ we_ref[...]
    o_ref[...] = jnp.maximum(a + b, 0.0)


def _add_body(a_ref, b_ref, o_ref):
    o_ref[...] = a_ref[...] + b_ref[...]


def _step_body(sc_ref, kd_ref, t_ref, e_ref, e0_ref, w4_ref, o_ref):
    a10, ir = sc_ref[0], sc_ref[1]
    e = e_ref[...]
    br, d4 = e.shape
    h = d4 // 4
    u = _uniform_at(kd_ref, (br, d4), pl.program_id(0) * br, d4, 0)
    rm = jnp.concatenate([e[:, h:2 * h], e[:, :h],
                          e[:, 3 * h:], e[:, 2 * h:3 * h]], axis=1)
    m = (t_ref[...] - rm) * _scale(u, a10, ir)
    o_ref[...] = jnp.maximum(e0_ref[...] + m @ w4_ref[...], 0.0)


def _final_body(s_ref, x_ref, pa_ref, pb_ref, wx_ref, wf_ref, o_ref):
    nd = x_ref.shape[1]
    s = s_ref[...]
    ff = pa_ref[...] + pb_ref[...]
    a = (x_ref[...] * s[:, :nd]) @ wx_ref[...]
    b = (ff * s[:, nd:]) @ wf_ref[...]
    o_ref[...] = jnp.maximum(a + b, 0.0)


def _rows_spec(b, d):
    return pl.BlockSpec((b, d), lambda i: (i, 0))


def _full_spec(shape):
    return pl.BlockSpec(shape, lambda i: tuple(0 for _ in shape))


_SMEM_SPEC = pl.BlockSpec(memory_space=pltpu.SMEM)


def _tc_init(s0, xg, ea, wx, we):
    E, nd = xg.shape
    ed = ea.shape[1]
    eh = wx.shape[1]
    B = 8000
    return pl.pallas_call(
        _init_body,
        grid=(E // B,),
        in_specs=[_rows_spec(B, nd + ed), _rows_spec(B, nd),
                  _rows_spec(B, ed), _full_spec((nd, eh)),
                  _full_spec((ed, eh))],
        out_specs=_rows_spec(B, eh),
        out_shape=jax.ShapeDtypeStruct((E, eh), jnp.float32),
    )(s0, xg, ea, wx, we)


def _tc_add(a, b):
    n, d = a.shape
    B = 5000
    return pl.pallas_call(
        _add_body,
        grid=(n // B,),
        in_specs=[_rows_spec(B, d), _rows_spec(B, d)],
        out_specs=_rows_spec(B, d),
        out_shape=jax.ShapeDtypeStruct((n, d), jnp.float32),
    )(a, b)


def _tc_step(sc, kd, t4, e4, e04, w4):
    r, d4 = t4.shape
    B = 4000
    return pl.pallas_call(
        _step_body,
        grid=(r // B,),
        in_specs=[_SMEM_SPEC, _SMEM_SPEC, _rows_spec(B, d4),
                  _rows_spec(B, d4), _rows_spec(B, d4), _full_spec((d4, d4))],
        out_specs=_rows_spec(B, d4),
        out_shape=jax.ShapeDtypeStruct((r, d4), jnp.float32),
    )(sc, kd, t4, e4, e04, w4)


def _tc_final(sl, x, pa, pb, wx, wf):
    n, nd = x.shape
    eh = pa.shape[1]
    nh = wx.shape[1]
    B = 5000
    return pl.pallas_call(
        _final_body,
        grid=(n // B,),
        in_specs=[_rows_spec(B, nd + eh), _rows_spec(B, nd),
                  _rows_spec(B, eh), _rows_spec(B, eh),
                  _full_spec((nd, nh)), _full_spec((eh, nh))],
        out_specs=_rows_spec(B, nh),
        out_shape=jax.ShapeDtypeStruct((n, nh), jnp.float32),
    )(sl, x, pa, pb, wx, wf)


def kernel(x, edge_attr, edge_index, W_init, W_eupd, W_last,
           p_init, p_eupd, p_last):
    n_nodes, nd = x.shape
    E, ed = edge_attr.shape
    eh = W_eupd.shape[0]
    n_steps = 3

    src = edge_index[0].astype(jnp.int32)
    dst = edge_index[1].astype(jnp.int32)
    src2d = src.reshape(E // IDX_W, IDX_W)
    dst2d = dst.reshape(E // IDX_W, IDX_W)

    nk = jax.random.key(1)

    def kd(i):
        return jax.random.key_data(jax.random.fold_in(nk, i))

    def scpair(p_logit):
        p = jax.nn.sigmoid(p_logit[0])
        a10 = ((p + EPS) / (1.0 - p + EPS)) ** 10
        ir = 1.0 / (1.0 - p)
        return jnp.stack([a10, ir]).astype(jnp.float32)

    sc0, sce, scl = scpair(p_init), scpair(p_eupd), scpair(p_last)

    zeros_n = jnp.zeros((n_nodes, eh), jnp.float32)
    zW = jnp.zeros_like(W_eupd)
    w4 = jnp.block([[W_eupd, zW, zW, zW], [zW, W_eupd, zW, zW],
                    [zW, zW, W_eupd, zW],
                    [zW, zW, zW, W_eupd]])

    gather_x = _make_sc_gather(nd, E)
    gather_h = _make_sc_gather(eh, E)
    scatter = _make_sc_scatter(n_nodes, E, eh)

    s0 = _tc_rng_scale(sc0, kd(0), E * (nd + ed)).reshape(E, nd + ed)
    sl = _tc_rng_scale(scl, kd(99),
                       n_nodes * (nd + eh)).reshape(n_nodes, nd + eh)

    xg = gather_x(x, src2d)
    e0 = _tc_init(s0, xg, edge_attr, W_init[:nd], W_init[nd:])
    e0r = e0.reshape(E // 4, 4 * eh)
    e_t = e0
    for i in range(n_steps):
        parts = scatter(e_t, dst2d, zeros_n)
        neigh = _tc_add(parts[:n_nodes], parts[n_nodes:])
        t = gather_h(neigh, src2d)
        e_t = _tc_step(sce, kd(10 + i), t.reshape(E // 4, 4 * eh),
                       e_t.reshape(E // 4, 4 * eh), e0r, w4)
        e_t = e_t.reshape(E, eh)
    parts = scatter(e_t, dst2d, zeros_n)
    return _tc_final(sl, x, parts[:n_nodes], parts[n_nodes:],
                     W_last[:nd], W_last[nd:])
